# baseline (device time: 27889 ns/iter reference)
import jax
import jax.numpy as jnp
from jax import lax
from jax.experimental import pallas as pl
from jax.experimental.pallas import tpu as pltpu

N_DEV = 4
B, SQ, SKV, DH = 2, 256, 256, 64
H_PER = 4
HD = H_PER * DH
D_MODEL = 512
ROWS = B * SQ
QR = ROWS // N_DEV


def kernel(x, Wq, K_ext, V_ext, Wo):
    x2 = x.reshape(ROWS, D_MODEL)
    K2 = K_ext.reshape(B, SKV, HD)
    V2 = V_ext.reshape(B, SKV, HD)

    def body(x_ref, wq_ref, k_ref, v_ref, wo_ref, out_ref,
             ctx_ref, rs_ref, rs_send, rs_recv, ag_send, ag_recv):
        my = lax.axis_index("i")

        barrier_sem = pltpu.get_barrier_semaphore()
        for r in range(1, N_DEV):
            pl.semaphore_signal(
                barrier_sem, inc=1,
                device_id=(lax.rem(my + r, N_DEV),),
                device_id_type=pl.DeviceIdType.MESH,
            )
        pl.semaphore_wait(barrier_sem, N_DEV - 1)

        wq = wq_ref[:, pl.ds(my * HD, HD)]
        q_all = jnp.dot(x_ref[...], wq, preferred_element_type=jnp.float32)

        qb = lax.broadcasted_iota(jnp.int32, (QR, QR), 0) // 64
        kb = lax.broadcasted_iota(jnp.int32, (QR, QR), 1) // 64
        mask = qb == kb

        rs_descs = []
        for c in range(N_DEV):
            b, h2 = divmod(c, 2)
            qc = q_all[c * QR:(c + 1) * QR, :]
            kc = k_ref[b, h2 * QR:(h2 + 1) * QR, :]
            vc = v_ref[b, h2 * QR:(h2 + 1) * QR, :]
            ctx_parts = []
            for h in range(H_PER):
                qh = qc[:, h * DH:(h + 1) * DH]
                kh = kc[:, h * DH:(h + 1) * DH]
                vh = vc[:, h * DH:(h + 1) * DH]
                s = lax.dot_general(
                    qh, kh, (((1,), (1,)), ((), ())),
                    preferred_element_type=jnp.float32,
                ) * 0.125
                s = jnp.where(mask, s, -1e9)
                s = s - jnp.max(s, axis=-1, keepdims=True)
                w = jnp.exp(s)
                w = w / jnp.sum(w, axis=-1, keepdims=True)
                ctx_parts.append(
                    jnp.dot(w, vh, preferred_element_type=jnp.float32))
            ctx_ref[pl.ds(c * QR, QR)] = jnp.concatenate(ctx_parts, axis=1)

            r = lax.rem(c - my + N_DEV, N_DEV)
            desc = pltpu.make_async_remote_copy(
                src_ref=ctx_ref.at[pl.ds(c * QR, QR)],
                dst_ref=rs_ref.at[r],
                send_sem=rs_send.at[r],
                recv_sem=rs_recv.at[r],
                device_id=(c,),
                device_id_type=pl.DeviceIdType.MESH,
            )
            rs_descs.append(desc)

            @pl.when(c != my)
            def _():
                desc.start()

        wo_mine = wo_ref[pl.ds(my * HD, HD), :]
        acc = jnp.dot(
            ctx_ref[pl.ds(my * QR, QR)], wo_mine,
            preferred_element_type=jnp.float32,
        )
        for r in (1, 3, 2):
            recv = pltpu.make_async_remote_copy(
                src_ref=ctx_ref.at[pl.ds(0, QR)],
                dst_ref=rs_ref.at[r],
                send_sem=rs_send.at[r],
                recv_sem=rs_recv.at[r],
                device_id=(my,),
                device_id_type=pl.DeviceIdType.MESH,
            )
            recv.wait_recv()
            sender = lax.rem(my + N_DEV - r, N_DEV)
            wo_s = wo_ref[pl.ds(sender * HD, HD), :]
            acc = acc + jnp.dot(
                rs_ref[r], wo_s, preferred_element_type=jnp.float32)
        out_ref[pl.ds(my * QR, QR)] = acc

        b_descs = []
        for r in (2, 1, 3):
            tgt = lax.rem(my + r, N_DEV)
            rdma = pltpu.make_async_remote_copy(
                src_ref=out_ref.at[pl.ds(my * QR, QR)],
                dst_ref=out_ref.at[pl.ds(my * QR, QR)],
                send_sem=ag_send.at[r],
                recv_sem=ag_recv.at[r],
                device_id=(tgt,),
                device_id_type=pl.DeviceIdType.MESH,
            )
            rdma.start()
            b_descs.append((r, rdma))
        for r, rdma in sorted(b_descs, key=lambda t: {1: 0, 3: 1, 2: 2}[t[0]]):
            rdma.wait_recv()

        for c in range(N_DEV):
            @pl.when(c != my)
            def _():
                rs_descs[c].wait_send()
        for _, rdma in b_descs:
            rdma.wait_send()

    out = pl.pallas_call(
        body,
        out_shape=jax.ShapeDtypeStruct((ROWS, D_MODEL), jnp.float32),
        in_specs=[pl.BlockSpec(memory_space=pltpu.VMEM)] * 5,
        out_specs=pl.BlockSpec(memory_space=pltpu.VMEM),
        scratch_shapes=[
            pltpu.VMEM((ROWS, HD), jnp.float32),
            pltpu.VMEM((N_DEV, QR, HD), jnp.float32),
            pltpu.SemaphoreType.DMA((N_DEV,)),
            pltpu.SemaphoreType.DMA((N_DEV,)),
            pltpu.SemaphoreType.DMA((N_DEV,)),
            pltpu.SemaphoreType.DMA((N_DEV,)),
        ],
        compiler_params=pltpu.CompilerParams(collective_id=0),
    )(x2, Wq, K2, V2, Wo)
    return out.reshape(B, SQ, D_MODEL)


# device time: 25400 ns/iter; 1.0980x vs baseline; 1.0980x over previous
import jax
import jax.numpy as jnp
from jax import lax
from jax.experimental import pallas as pl
from jax.experimental.pallas import tpu as pltpu

N_DEV = 4
B, SQ, SKV, DH = 2, 256, 256, 64
H_PER = 4
HD = H_PER * DH
D_MODEL = 512
ROWS = B * SQ
QR = ROWS // N_DEV


def kernel(x, Wq, K_ext, V_ext, Wo):
    x2 = x.reshape(ROWS, D_MODEL)
    K2 = K_ext.reshape(B, SKV, HD)
    V2 = V_ext.reshape(B, SKV, HD)

    def body(x_ref, wq_ref, k_ref, v_ref, wo_ref, out_ref,
             ctx_ref, rs_ref, rs_send, rs_recv, ag_send, ag_recv):
        my = lax.axis_index("i")

        barrier_sem = pltpu.get_barrier_semaphore()
        for r in range(1, N_DEV):
            pl.semaphore_signal(
                barrier_sem, inc=1,
                device_id=(lax.rem(my + r, N_DEV),),
                device_id_type=pl.DeviceIdType.MESH,
            )
        pl.semaphore_wait(barrier_sem, N_DEV - 1)

        wq = wq_ref[:, pl.ds(my * HD, HD)]
        q_all = jnp.dot(x_ref[...], wq, preferred_element_type=jnp.float32)

        qb = lax.broadcasted_iota(jnp.int32, (SQ, SKV), 0) // 64
        kb = lax.broadcasted_iota(jnp.int32, (SQ, SKV), 1) // 64
        mask = (qb == kb) | ((kb % 4) == (qb % 4))

        rs_descs = [None] * N_DEV
        for b in range(B):
            q = q_all[b * SQ:(b + 1) * SQ, :]
            kbat = k_ref[b]
            vbat = v_ref[b]
            ctx_parts = []
            for h in range(H_PER):
                qh = q[:, h * DH:(h + 1) * DH]
                kh = kbat[:, h * DH:(h + 1) * DH]
                vh = vbat[:, h * DH:(h + 1) * DH]
                s = lax.dot_general(
                    qh, kh, (((1,), (1,)), ((), ())),
                    preferred_element_type=jnp.float32,
                ) * 0.125
                s = jnp.where(mask, s, -1e9)
                s = s - jnp.max(s, axis=-1, keepdims=True)
                w = jnp.exp(s)
                w = w / jnp.sum(w, axis=-1, keepdims=True)
                ctx_parts.append(
                    jnp.dot(w, vh, preferred_element_type=jnp.float32))
            ctx_ref[pl.ds(b * SQ, SQ)] = jnp.concatenate(ctx_parts, axis=1)

            for c in (2 * b, 2 * b + 1):
                r = lax.rem(c - my + N_DEV, N_DEV)
                desc = pltpu.make_async_remote_copy(
                    src_ref=ctx_ref.at[pl.ds(c * QR, QR)],
                    dst_ref=rs_ref.at[r],
                    send_sem=rs_send.at[r],
                    recv_sem=rs_recv.at[r],
                    device_id=(c,),
                    device_id_type=pl.DeviceIdType.MESH,
                )
                rs_descs[c] = desc

                @pl.when(c != my)
                def _():
                    desc.start()

        wo_mine = wo_ref[pl.ds(my * HD, HD), :]
        acc = jnp.dot(
            ctx_ref[pl.ds(my * QR, QR)], wo_mine,
            preferred_element_type=jnp.float32,
        )
        for r in (1, 3, 2):
            recv = pltpu.make_async_remote_copy(
                src_ref=ctx_ref.at[pl.ds(0, QR)],
                dst_ref=rs_ref.at[r],
                send_sem=rs_send.at[r],
                recv_sem=rs_recv.at[r],
                device_id=(my,),
                device_id_type=pl.DeviceIdType.MESH,
            )
            recv.wait_recv()
            sender = lax.rem(my + N_DEV - r, N_DEV)
            wo_s = wo_ref[pl.ds(sender * HD, HD), :]
            acc = acc + jnp.dot(
                rs_ref[r], wo_s, preferred_element_type=jnp.float32)
        out_ref[pl.ds(my * QR, QR)] = acc

        b_descs = {}
        for r in (2, 1, 3):
            tgt = lax.rem(my + r, N_DEV)
            rdma = pltpu.make_async_remote_copy(
                src_ref=out_ref.at[pl.ds(my * QR, QR)],
                dst_ref=out_ref.at[pl.ds(my * QR, QR)],
                send_sem=ag_send.at[r],
                recv_sem=ag_recv.at[r],
                device_id=(tgt,),
                device_id_type=pl.DeviceIdType.MESH,
            )
            rdma.start()
            b_descs[r] = rdma
        for r in (1, 3, 2):
            b_descs[r].wait_recv()

        for c in range(N_DEV):
            @pl.when(c != my)
            def _():
                rs_descs[c].wait_send()
        for r in (1, 2, 3):
            b_descs[r].wait_send()

    out = pl.pallas_call(
        body,
        out_shape=jax.ShapeDtypeStruct((ROWS, D_MODEL), jnp.float32),
        in_specs=[pl.BlockSpec(memory_space=pltpu.VMEM)] * 5,
        out_specs=pl.BlockSpec(memory_space=pltpu.VMEM),
        scratch_shapes=[
            pltpu.VMEM((ROWS, HD), jnp.float32),
            pltpu.VMEM((N_DEV, QR, HD), jnp.float32),
            pltpu.SemaphoreType.DMA((N_DEV,)),
            pltpu.SemaphoreType.DMA((N_DEV,)),
            pltpu.SemaphoreType.DMA((N_DEV,)),
            pltpu.SemaphoreType.DMA((N_DEV,)),
        ],
        compiler_params=pltpu.CompilerParams(collective_id=0),
    )(x2, Wq, K2, V2, Wo)
    return out.reshape(B, SQ, D_MODEL)


# device time: 21657 ns/iter; 1.2878x vs baseline; 1.1728x over previous
import jax
import jax.numpy as jnp
from jax import lax
from jax.experimental import pallas as pl
from jax.experimental.pallas import tpu as pltpu

N_DEV = 4
B, SQ, SKV, DH = 2, 256, 256, 64
H_PER = 4
HD = H_PER * DH
D_MODEL = 512
ROWS = B * SQ
QR = ROWS // N_DEV


def kernel(x, Wq, K_ext, V_ext, Wo):
    x2 = x.reshape(ROWS, D_MODEL)
    K2 = K_ext.reshape(B, SKV, HD)
    V2 = V_ext.reshape(B, SKV, HD)

    def body(x_ref, wq_ref, k_ref, v_ref, wo_ref, out_ref,
             ctx_ref, rs_ref, agsrc_ref, ag_ref,
             rs_send, rs_recv, ag_send, ag_recv):
        my = lax.axis_index("i")

        barrier_sem = pltpu.get_barrier_semaphore()
        for r in range(1, N_DEV):
            pl.semaphore_signal(
                barrier_sem, inc=1,
                device_id=(lax.rem(my + r, N_DEV),),
                device_id_type=pl.DeviceIdType.MESH,
            )
        pl.semaphore_wait(barrier_sem, N_DEV - 1)

        wq = wq_ref[:, pl.ds(my * HD, HD)]
        q_all = jnp.dot(x_ref[...], wq, preferred_element_type=jnp.float32)
        q_all = q_all * 0.125

        qb = lax.broadcasted_iota(jnp.int32, (SQ, SKV), 0) // 64
        kb = lax.broadcasted_iota(jnp.int32, (SQ, SKV), 1) // 64
        maskf = jnp.where(
            (qb == kb) | ((kb % 4) == (qb % 4)), 1.0, 0.0
        ).astype(jnp.float32)

        rs_descs = [None] * N_DEV
        for b in range(B):
            q = q_all[b * SQ:(b + 1) * SQ, :]
            kbat = k_ref[b]
            vbat = v_ref[b]
            ctx_parts = []
            for h in range(H_PER):
                qh = q[:, h * DH:(h + 1) * DH]
                kh = kbat[:, h * DH:(h + 1) * DH]
                vh = vbat[:, h * DH:(h + 1) * DH]
                s = lax.dot_general(
                    qh, kh, (((1,), (1,)), ((), ())),
                    preferred_element_type=jnp.float32,
                )
                w = jnp.exp(s) * maskf
                rs = 1.0 / jnp.sum(w, axis=-1, keepdims=True)
                ctx_parts.append(
                    jnp.dot(w, vh, preferred_element_type=jnp.float32) * rs)
            ctx_ref[pl.ds(b * SQ, SQ)] = jnp.concatenate(ctx_parts, axis=1)

            for c in (2 * b, 2 * b + 1):
                r = lax.rem(c - my + N_DEV, N_DEV)
                desc = pltpu.make_async_remote_copy(
                    src_ref=ctx_ref.at[pl.ds(c * QR, QR)],
                    dst_ref=rs_ref.at[r],
                    send_sem=rs_send.at[r],
                    recv_sem=rs_recv.at[r],
                    device_id=(c,),
                    device_id_type=pl.DeviceIdType.MESH,
                )
                rs_descs[c] = desc

                @pl.when(c != my)
                def _():
                    desc.start()

        wo_mine = wo_ref[pl.ds(my * HD, HD), :]
        acc = jnp.dot(
            ctx_ref[pl.ds(my * QR, QR)], wo_mine,
            preferred_element_type=jnp.float32,
        )
        for r in (1, 3, 2):
            recv = pltpu.make_async_remote_copy(
                src_ref=ctx_ref.at[pl.ds(0, QR)],
                dst_ref=rs_ref.at[r],
                send_sem=rs_send.at[r],
                recv_sem=rs_recv.at[r],
                device_id=(my,),
                device_id_type=pl.DeviceIdType.MESH,
            )
            recv.wait_recv()
            sender = lax.rem(my + N_DEV - r, N_DEV)
            wo_s = wo_ref[pl.ds(sender * HD, HD), :]
            acc = acc + jnp.dot(
                rs_ref[r], wo_s, preferred_element_type=jnp.float32)
        out_ref[pl.ds(my * QR, QR)] = acc
        agsrc_ref[...] = acc.astype(jnp.bfloat16)

        b_descs = {}
        for r in (2, 1, 3):
            tgt = lax.rem(my + r, N_DEV)
            rdma = pltpu.make_async_remote_copy(
                src_ref=agsrc_ref,
                dst_ref=ag_ref.at[r],
                send_sem=ag_send.at[r],
                recv_sem=ag_recv.at[r],
                device_id=(tgt,),
                device_id_type=pl.DeviceIdType.MESH,
            )
            rdma.start()
            b_descs[r] = rdma
        for r in (1, 3, 2):
            b_descs[r].wait_recv()
            sender = lax.rem(my + N_DEV - r, N_DEV)
            out_ref[pl.ds(sender * QR, QR)] = ag_ref[r].astype(jnp.float32)

        for c in range(N_DEV):
            @pl.when(c != my)
            def _():
                rs_descs[c].wait_send()
        for r in (1, 2, 3):
            b_descs[r].wait_send()

    out = pl.pallas_call(
        body,
        out_shape=jax.ShapeDtypeStruct((ROWS, D_MODEL), jnp.float32),
        in_specs=[pl.BlockSpec(memory_space=pltpu.VMEM)] * 5,
        out_specs=pl.BlockSpec(memory_space=pltpu.VMEM),
        scratch_shapes=[
            pltpu.VMEM((ROWS, HD), jnp.float32),
            pltpu.VMEM((N_DEV, QR, HD), jnp.float32),
            pltpu.VMEM((QR, D_MODEL), jnp.bfloat16),
            pltpu.VMEM((N_DEV, QR, D_MODEL), jnp.bfloat16),
            pltpu.SemaphoreType.DMA((N_DEV,)),
            pltpu.SemaphoreType.DMA((N_DEV,)),
            pltpu.SemaphoreType.DMA((N_DEV,)),
            pltpu.SemaphoreType.DMA((N_DEV,)),
        ],
        compiler_params=pltpu.CompilerParams(collective_id=0),
    )(x2, Wq, K2, V2, Wo)
    return out.reshape(B, SQ, D_MODEL)


# device time: 19962 ns/iter; 1.3971x vs baseline; 1.0849x over previous
import jax
import jax.numpy as jnp
from jax import lax
from jax.experimental import pallas as pl
from jax.experimental.pallas import tpu as pltpu

N_DEV = 4
B, SQ, SKV, DH = 2, 256, 256, 64
H_PER = 4
HD = H_PER * DH
D_MODEL = 512
ROWS = B * SQ
QR = ROWS // N_DEV


def kernel(x, Wq, K_ext, V_ext, Wo):
    x2 = x.reshape(ROWS, D_MODEL)
    K2 = K_ext.reshape(B, SKV, HD)
    V2 = V_ext.reshape(B, SKV, HD)

    def body(x_ref, wq_ref, k_ref, v_ref, wo_ref, out_ref,
             ctx_ref, rs_ref, agsrc_ref, ag_ref, wobf_ref,
             rs_send, rs_recv, ag_send, ag_recv):
        my = lax.axis_index("i")

        barrier_sem = pltpu.get_barrier_semaphore()
        for r in range(1, N_DEV):
            pl.semaphore_signal(
                barrier_sem, inc=1,
                device_id=(lax.rem(my + r, N_DEV),),
                device_id_type=pl.DeviceIdType.MESH,
            )
        pl.semaphore_wait(barrier_sem, N_DEV - 1)

        wobf_ref[...] = wo_ref[...].astype(jnp.bfloat16)

        wq = wq_ref[:, pl.ds(my * HD, HD)]
        q_all = jnp.dot(x_ref[...], wq, preferred_element_type=jnp.float32)
        q_all = q_all * 0.125

        qb = lax.broadcasted_iota(jnp.int32, (SQ, SKV), 0) // 64
        kb = lax.broadcasted_iota(jnp.int32, (SQ, SKV), 1) // 64
        maskf = jnp.where(
            (qb == kb) | ((kb % 4) == (qb % 4)), 1.0, 0.0
        ).astype(jnp.float32)

        rs_descs = [None] * N_DEV
        for b in range(B):
            q = q_all[b * SQ:(b + 1) * SQ, :]
            kbat = k_ref[b]
            vbat = v_ref[b]
            ctx_parts = []
            for h in range(H_PER):
                qh = q[:, h * DH:(h + 1) * DH]
                kh = kbat[:, h * DH:(h + 1) * DH]
                vh = vbat[:, h * DH:(h + 1) * DH]
                s = lax.dot_general(
                    qh, kh, (((1,), (1,)), ((), ())),
                    preferred_element_type=jnp.float32,
                )
                w = jnp.exp(s) * maskf
                rs = 1.0 / jnp.sum(w, axis=-1, keepdims=True)
                ctx_parts.append(
                    jnp.dot(w, vh, preferred_element_type=jnp.float32) * rs)
            ctx_ref[pl.ds(b * SQ, SQ)] = jnp.concatenate(
                ctx_parts, axis=1).astype(jnp.bfloat16)

            for c in (2 * b, 2 * b + 1):
                r = lax.rem(c - my + N_DEV, N_DEV)
                desc = pltpu.make_async_remote_copy(
                    src_ref=ctx_ref.at[pl.ds(c * QR, QR)],
                    dst_ref=rs_ref.at[r],
                    send_sem=rs_send.at[r],
                    recv_sem=rs_recv.at[r],
                    device_id=(c,),
                    device_id_type=pl.DeviceIdType.MESH,
                )
                rs_descs[c] = desc

                @pl.when(c != my)
                def _():
                    desc.start()

        wo_mine = wobf_ref[pl.ds(my * HD, HD), :]
        acc = jnp.dot(
            ctx_ref[pl.ds(my * QR, QR)], wo_mine,
            preferred_element_type=jnp.float32,
        )
        for r in (1, 3, 2):
            recv = pltpu.make_async_remote_copy(
                src_ref=ctx_ref.at[pl.ds(0, QR)],
                dst_ref=rs_ref.at[r],
                send_sem=rs_send.at[r],
                recv_sem=rs_recv.at[r],
                device_id=(my,),
                device_id_type=pl.DeviceIdType.MESH,
            )
            recv.wait_recv()
            sender = lax.rem(my + N_DEV - r, N_DEV)
            wo_s = wobf_ref[pl.ds(sender * HD, HD), :]
            acc = acc + jnp.dot(
                rs_ref[r], wo_s, preferred_element_type=jnp.float32)
        out_ref[pl.ds(my * QR, QR)] = acc
        agsrc_ref[...] = acc.astype(jnp.bfloat16)

        b_descs = {}
        for r in (2, 1, 3):
            tgt = lax.rem(my + r, N_DEV)
            rdma = pltpu.make_async_remote_copy(
                src_ref=agsrc_ref,
                dst_ref=ag_ref.at[r],
                send_sem=ag_send.at[r],
                recv_sem=ag_recv.at[r],
                device_id=(tgt,),
                device_id_type=pl.DeviceIdType.MESH,
            )
            rdma.start()
            b_descs[r] = rdma
        for r in (1, 3, 2):
            b_descs[r].wait_recv()
            sender = lax.rem(my + N_DEV - r, N_DEV)
            out_ref[pl.ds(sender * QR, QR)] = ag_ref[r].astype(jnp.float32)

        for c in range(N_DEV):
            @pl.when(c != my)
            def _():
                rs_descs[c].wait_send()
        for r in (1, 2, 3):
            b_descs[r].wait_send()

    out = pl.pallas_call(
        body,
        out_shape=jax.ShapeDtypeStruct((ROWS, D_MODEL), jnp.float32),
        in_specs=[pl.BlockSpec(memory_space=pltpu.VMEM)] * 5,
        out_specs=pl.BlockSpec(memory_space=pltpu.VMEM),
        scratch_shapes=[
            pltpu.VMEM((ROWS, HD), jnp.bfloat16),
            pltpu.VMEM((N_DEV, QR, HD), jnp.bfloat16),
            pltpu.VMEM((QR, D_MODEL), jnp.bfloat16),
            pltpu.VMEM((N_DEV, QR, D_MODEL), jnp.bfloat16),
            pltpu.VMEM((HD * N_DEV, D_MODEL), jnp.bfloat16),
            pltpu.SemaphoreType.DMA((N_DEV,)),
            pltpu.SemaphoreType.DMA((N_DEV,)),
            pltpu.SemaphoreType.DMA((N_DEV,)),
            pltpu.SemaphoreType.DMA((N_DEV,)),
        ],
        compiler_params=pltpu.CompilerParams(collective_id=0),
    )(x2, Wq, K2, V2, Wo)
    return out.reshape(B, SQ, D_MODEL)


# device time: 18825 ns/iter; 1.4815x vs baseline; 1.0604x over previous
import jax
import jax.numpy as jnp
from jax import lax
from jax.experimental import pallas as pl
from jax.experimental.pallas import tpu as pltpu

N_DEV = 4
B, SQ, SKV, DH = 2, 256, 256, 64
H_PER = 4
HD = H_PER * DH
D_MODEL = 512
ROWS = B * SQ
QR = ROWS // N_DEV


def kernel(x, Wq, K_ext, V_ext, Wo):
    x2 = x.reshape(ROWS, D_MODEL)
    K2 = K_ext.reshape(B, SKV, HD)
    V2 = V_ext.reshape(B, SKV, HD)

    def body(x_ref, wq_ref, k_ref, v_ref, wo_ref, out_ref,
             ctx_ref, rs_ref, agsrc_ref, ag_ref, wobf_ref,
             rs_send, rs_recv, ag_send, ag_recv):
        my = lax.axis_index("i")

        barrier_sem = pltpu.get_barrier_semaphore()
        for r in range(1, N_DEV):
            pl.semaphore_signal(
                barrier_sem, inc=1,
                device_id=(lax.rem(my + r, N_DEV),),
                device_id_type=pl.DeviceIdType.MESH,
            )

        wobf_ref[...] = wo_ref[...].astype(jnp.bfloat16)

        wq = wq_ref[:, pl.ds(my * HD, HD)]
        q_all = jnp.dot(x_ref[...], wq, preferred_element_type=jnp.float32)
        q_all = q_all * 0.125

        qb = lax.broadcasted_iota(jnp.int32, (SQ, SKV), 0) // 64
        kb = lax.broadcasted_iota(jnp.int32, (SQ, SKV), 1) // 64
        maskf = jnp.where(
            (qb == kb) | ((kb % 4) == (qb % 4)), 1.0, 0.0
        ).astype(jnp.float32)

        rs_descs = [None] * N_DEV
        for b in range(B):
            q = q_all[b * SQ:(b + 1) * SQ, :]
            kbat = k_ref[b]
            vbat = v_ref[b]
            ctx_parts = []
            for h in range(H_PER):
                qh = q[:, h * DH:(h + 1) * DH]
                kh = kbat[:, h * DH:(h + 1) * DH]
                vh = vbat[:, h * DH:(h + 1) * DH]
                s = lax.dot_general(
                    qh, kh, (((1,), (1,)), ((), ())),
                    preferred_element_type=jnp.float32,
                )
                w = jnp.exp(s) * maskf
                rs = 1.0 / jnp.sum(w, axis=-1, keepdims=True)
                ctx_parts.append(
                    jnp.dot(w, vh, preferred_element_type=jnp.float32) * rs)
            ctx_ref[pl.ds(b * SQ, SQ)] = jnp.concatenate(
                ctx_parts, axis=1).astype(jnp.bfloat16)

            if b == 0:
                pl.semaphore_wait(barrier_sem, N_DEV - 1)

            for c in (2 * b, 2 * b + 1):
                r = lax.rem(c - my + N_DEV, N_DEV)
                desc = pltpu.make_async_remote_copy(
                    src_ref=ctx_ref.at[pl.ds(c * QR, QR)],
                    dst_ref=rs_ref.at[r],
                    send_sem=rs_send.at[r],
                    recv_sem=rs_recv.at[r],
                    device_id=(c,),
                    device_id_type=pl.DeviceIdType.MESH,
                )
                rs_descs[c] = desc

                @pl.when(c != my)
                def _():
                    desc.start()

        wo_mine = wobf_ref[pl.ds(my * HD, HD), :]
        acc = jnp.dot(
            ctx_ref[pl.ds(my * QR, QR)], wo_mine,
            preferred_element_type=jnp.float32,
        )
        for r in (1, 3, 2):
            recv = pltpu.make_async_remote_copy(
                src_ref=ctx_ref.at[pl.ds(0, QR)],
                dst_ref=rs_ref.at[r],
                send_sem=rs_send.at[r],
                recv_sem=rs_recv.at[r],
                device_id=(my,),
                device_id_type=pl.DeviceIdType.MESH,
            )
            recv.wait_recv()
            sender = lax.rem(my + N_DEV - r, N_DEV)
            wo_s = wobf_ref[pl.ds(sender * HD, HD), :]
            acc = acc + jnp.dot(
                rs_ref[r], wo_s, preferred_element_type=jnp.float32)
        out_ref[pl.ds(my * QR, QR)] = acc
        agsrc_ref[...] = acc.astype(jnp.bfloat16)

        b_descs = {}
        for r in (2, 1, 3):
            tgt = lax.rem(my + r, N_DEV)
            rdma = pltpu.make_async_remote_copy(
                src_ref=agsrc_ref,
                dst_ref=ag_ref.at[r],
                send_sem=ag_send.at[r],
                recv_sem=ag_recv.at[r],
                device_id=(tgt,),
                device_id_type=pl.DeviceIdType.MESH,
            )
            rdma.start()
            b_descs[r] = rdma
        for r in (1, 3, 2):
            b_descs[r].wait_recv()
            sender = lax.rem(my + N_DEV - r, N_DEV)
            out_ref[pl.ds(sender * QR, QR)] = ag_ref[r].astype(jnp.float32)

        for c in range(N_DEV):
            @pl.when(c != my)
            def _():
                rs_descs[c].wait_send()
        for r in (1, 2, 3):
            b_descs[r].wait_send()

    out = pl.pallas_call(
        body,
        out_shape=jax.ShapeDtypeStruct((ROWS, D_MODEL), jnp.float32),
        in_specs=[pl.BlockSpec(memory_space=pltpu.VMEM)] * 5,
        out_specs=pl.BlockSpec(memory_space=pltpu.VMEM),
        scratch_shapes=[
            pltpu.VMEM((ROWS, HD), jnp.bfloat16),
            pltpu.VMEM((N_DEV, QR, HD), jnp.bfloat16),
            pltpu.VMEM((QR, D_MODEL), jnp.bfloat16),
            pltpu.VMEM((N_DEV, QR, D_MODEL), jnp.bfloat16),
            pltpu.VMEM((HD * N_DEV, D_MODEL), jnp.bfloat16),
            pltpu.SemaphoreType.DMA((N_DEV,)),
            pltpu.SemaphoreType.DMA((N_DEV,)),
            pltpu.SemaphoreType.DMA((N_DEV,)),
            pltpu.SemaphoreType.DMA((N_DEV,)),
        ],
        compiler_params=pltpu.CompilerParams(collective_id=0),
    )(x2, Wq, K2, V2, Wo)
    return out.reshape(B, SQ, D_MODEL)


# device time: 18428 ns/iter; 1.5134x vs baseline; 1.0215x over previous
import jax
import jax.numpy as jnp
from jax import lax
from jax.experimental import pallas as pl
from jax.experimental.pallas import tpu as pltpu

N_DEV = 4
B, SQ, SKV, DH = 2, 256, 256, 64
H_PER = 4
HD = H_PER * DH
D_MODEL = 512
ROWS = B * SQ
QR = ROWS // N_DEV


def kernel(x, Wq, K_ext, V_ext, Wo):
    def body(x_ref, wq_ref, k_ref, v_ref, wo_ref, out_ref,
             ctx_ref, rs_ref, agsrc_ref, ag_ref, wobf_ref,
             rs_send, rs_recv, ag_send, ag_recv):
        my = lax.axis_index("i")

        barrier_sem = pltpu.get_barrier_semaphore()
        for r in range(1, N_DEV):
            pl.semaphore_signal(
                barrier_sem, inc=1,
                device_id=(lax.rem(my + r, N_DEV),),
                device_id_type=pl.DeviceIdType.MESH,
            )

        wobf_ref[...] = wo_ref[...].astype(jnp.bfloat16)

        wq = wq_ref[:, pl.ds(my * HD, HD)]

        qb = lax.broadcasted_iota(jnp.int32, (SQ, SKV), 0) // 64
        kb = lax.broadcasted_iota(jnp.int32, (SQ, SKV), 1) // 64
        maskf = jnp.where(
            (qb == kb) | ((kb % 4) == (qb % 4)), 1.0, 0.0
        ).astype(jnp.float32)

        rs_descs = [None] * N_DEV
        for b in range(B):
            q = jnp.dot(x_ref[b], wq, preferred_element_type=jnp.float32)
            q = q * 0.125
            ctx_parts = []
            for h in range(H_PER):
                qh = q[:, h * DH:(h + 1) * DH]
                kh = k_ref[b, :, h, :]
                vh = v_ref[b, :, h, :]
                s = lax.dot_general(
                    qh, kh, (((1,), (1,)), ((), ())),
                    preferred_element_type=jnp.float32,
                )
                w = jnp.exp(s) * maskf
                rs = 1.0 / jnp.sum(w, axis=-1, keepdims=True)
                ctx_parts.append(
                    jnp.dot(w, vh, preferred_element_type=jnp.float32) * rs)
            ctx_ref[pl.ds(b * SQ, SQ)] = jnp.concatenate(
                ctx_parts, axis=1).astype(jnp.bfloat16)

            if b == 0:
                pl.semaphore_wait(barrier_sem, N_DEV - 1)

            for c in (2 * b, 2 * b + 1):
                r = lax.rem(c - my + N_DEV, N_DEV)
                desc = pltpu.make_async_remote_copy(
                    src_ref=ctx_ref.at[pl.ds(c * QR, QR)],
                    dst_ref=rs_ref.at[r],
                    send_sem=rs_send.at[r],
                    recv_sem=rs_recv.at[r],
                    device_id=(c,),
                    device_id_type=pl.DeviceIdType.MESH,
                )
                rs_descs[c] = desc

                @pl.when(c != my)
                def _():
                    desc.start()

        wo_mine = wobf_ref[pl.ds(my * HD, HD), :]
        acc = jnp.dot(
            ctx_ref[pl.ds(my * QR, QR)], wo_mine,
            preferred_element_type=jnp.float32,
        )
        for r in (1, 3, 2):
            recv = pltpu.make_async_remote_copy(
                src_ref=ctx_ref.at[pl.ds(0, QR)],
                dst_ref=rs_ref.at[r],
                send_sem=rs_send.at[r],
                recv_sem=rs_recv.at[r],
                device_id=(my,),
                device_id_type=pl.DeviceIdType.MESH,
            )
            recv.wait_recv()
            sender = lax.rem(my + N_DEV - r, N_DEV)
            wo_s = wobf_ref[pl.ds(sender * HD, HD), :]
            acc = acc + jnp.dot(
                rs_ref[r], wo_s, preferred_element_type=jnp.float32)
        out_ref[my // 2, pl.ds((my % 2) * QR, QR)] = acc
        agsrc_ref[...] = acc.astype(jnp.bfloat16)

        b_descs = {}
        for r in (2, 1, 3):
            tgt = lax.rem(my + r, N_DEV)
            rdma = pltpu.make_async_remote_copy(
                src_ref=agsrc_ref,
                dst_ref=ag_ref.at[r],
                send_sem=ag_send.at[r],
                recv_sem=ag_recv.at[r],
                device_id=(tgt,),
                device_id_type=pl.DeviceIdType.MESH,
            )
            rdma.start()
            b_descs[r] = rdma
        for r in (1, 3, 2):
            b_descs[r].wait_recv()
            sender = lax.rem(my + N_DEV - r, N_DEV)
            out_ref[sender // 2, pl.ds((sender % 2) * QR, QR)] = (
                ag_ref[r].astype(jnp.float32))

        for c in range(N_DEV):
            @pl.when(c != my)
            def _():
                rs_descs[c].wait_send()
        for r in (1, 2, 3):
            b_descs[r].wait_send()

    return pl.pallas_call(
        body,
        out_shape=jax.ShapeDtypeStruct((B, SQ, D_MODEL), jnp.float32),
        in_specs=[pl.BlockSpec(memory_space=pltpu.VMEM)] * 5,
        out_specs=pl.BlockSpec(memory_space=pltpu.VMEM),
        scratch_shapes=[
            pltpu.VMEM((ROWS, HD), jnp.bfloat16),
            pltpu.VMEM((N_DEV, QR, HD), jnp.bfloat16),
            pltpu.VMEM((QR, D_MODEL), jnp.bfloat16),
            pltpu.VMEM((N_DEV, QR, D_MODEL), jnp.bfloat16),
            pltpu.VMEM((HD * N_DEV, D_MODEL), jnp.bfloat16),
            pltpu.SemaphoreType.DMA((N_DEV,)),
            pltpu.SemaphoreType.DMA((N_DEV,)),
            pltpu.SemaphoreType.DMA((N_DEV,)),
            pltpu.SemaphoreType.DMA((N_DEV,)),
        ],
        compiler_params=pltpu.CompilerParams(collective_id=0),
    )(x, Wq, K_ext, V_ext, Wo)


# device time: 18395 ns/iter; 1.5161x vs baseline; 1.0018x over previous
import jax
import jax.numpy as jnp
from jax import lax
from jax.experimental import pallas as pl
from jax.experimental.pallas import tpu as pltpu

N_DEV = 4
B, SQ, SKV, DH = 2, 256, 256, 64
H_PER = 4
HD = H_PER * DH
D_MODEL = 512
ROWS = B * SQ
QR = ROWS // N_DEV


def kernel(x, Wq, K_ext, V_ext, Wo):
    def body(x_ref, wq_ref, k_ref, v_ref, wo_ref, out_ref,
             ctx_ref, rs_ref, agsrc_ref, ag_ref, wobf_ref,
             rs_send, rs_recv, ag_send, ag_recv):
        my = lax.axis_index("i")

        barrier_sem = pltpu.get_barrier_semaphore()
        for r in range(1, N_DEV):
            pl.semaphore_signal(
                barrier_sem, inc=1,
                device_id=(lax.rem(my + r, N_DEV),),
                device_id_type=pl.DeviceIdType.MESH,
            )

        wq = wq_ref[:, pl.ds(my * HD, HD)]

        qb = lax.broadcasted_iota(jnp.int32, (SQ, SKV), 0) // 64
        kb = lax.broadcasted_iota(jnp.int32, (SQ, SKV), 1) // 64
        maskf = jnp.where(
            (qb == kb) | ((kb % 4) == (qb % 4)), 1.0, 0.0
        ).astype(jnp.float32)

        rs_descs = [None] * N_DEV
        for b in range(B):
            q = jnp.dot(x_ref[b], wq, preferred_element_type=jnp.float32)
            q = q * 0.125
            ctx_parts = []
            for h in range(H_PER):
                qh = q[:, h * DH:(h + 1) * DH]
                kh = k_ref[b, :, h, :]
                vh = v_ref[b, :, h, :]
                s = lax.dot_general(
                    qh, kh, (((1,), (1,)), ((), ())),
                    preferred_element_type=jnp.float32,
                )
                w = jnp.exp(s) * maskf
                rs = 1.0 / jnp.sum(w, axis=-1, keepdims=True)
                ctx_parts.append(
                    jnp.dot(w, vh, preferred_element_type=jnp.float32) * rs)
            ctx_ref[pl.ds(b * SQ, SQ)] = jnp.concatenate(
                ctx_parts, axis=1).astype(jnp.bfloat16)

            if b == 0:
                pl.semaphore_wait(barrier_sem, N_DEV - 1)

            for c in (2 * b, 2 * b + 1):
                r = lax.rem(c - my + N_DEV, N_DEV)
                desc = pltpu.make_async_remote_copy(
                    src_ref=ctx_ref.at[pl.ds(c * QR, QR)],
                    dst_ref=rs_ref.at[r],
                    send_sem=rs_send.at[r],
                    recv_sem=rs_recv.at[r],
                    device_id=(c,),
                    device_id_type=pl.DeviceIdType.MESH,
                )
                rs_descs[c] = desc

                @pl.when(c != my)
                def _():
                    desc.start()

        wobf_ref[...] = wo_ref[...].astype(jnp.bfloat16)
        wo_mine = wobf_ref[pl.ds(my * HD, HD), :]
        acc = jnp.dot(
            ctx_ref[pl.ds(my * QR, QR)], wo_mine,
            preferred_element_type=jnp.float32,
        )
        for r in (1, 3, 2):
            recv = pltpu.make_async_remote_copy(
                src_ref=ctx_ref.at[pl.ds(0, QR)],
                dst_ref=rs_ref.at[r],
                send_sem=rs_send.at[r],
                recv_sem=rs_recv.at[r],
                device_id=(my,),
                device_id_type=pl.DeviceIdType.MESH,
            )
            recv.wait_recv()
            sender = lax.rem(my + N_DEV - r, N_DEV)
            wo_s = wobf_ref[pl.ds(sender * HD, HD), :]
            acc = acc + jnp.dot(
                rs_ref[r], wo_s, preferred_element_type=jnp.float32)
        agsrc_ref[...] = acc.astype(jnp.bfloat16)

        b_descs = {}
        for r in (2, 1, 3):
            tgt = lax.rem(my + r, N_DEV)
            rdma = pltpu.make_async_remote_copy(
                src_ref=agsrc_ref,
                dst_ref=ag_ref.at[r],
                send_sem=ag_send.at[r],
                recv_sem=ag_recv.at[r],
                device_id=(tgt,),
                device_id_type=pl.DeviceIdType.MESH,
            )
            rdma.start()
            b_descs[r] = rdma
        out_ref[my // 2, pl.ds((my % 2) * QR, QR)] = acc
        for r in (1, 3, 2):
            b_descs[r].wait_recv()
            sender = lax.rem(my + N_DEV - r, N_DEV)
            out_ref[sender // 2, pl.ds((sender % 2) * QR, QR)] = (
                ag_ref[r].astype(jnp.float32))

        for c in range(N_DEV):
            @pl.when(c != my)
            def _():
                rs_descs[c].wait_send()
        for r in (1, 2, 3):
            b_descs[r].wait_send()

    return pl.pallas_call(
        body,
        out_shape=jax.ShapeDtypeStruct((B, SQ, D_MODEL), jnp.float32),
        in_specs=[pl.BlockSpec(memory_space=pltpu.VMEM)] * 5,
        out_specs=pl.BlockSpec(memory_space=pltpu.VMEM),
        scratch_shapes=[
            pltpu.VMEM((ROWS, HD), jnp.bfloat16),
            pltpu.VMEM((N_DEV, QR, HD), jnp.bfloat16),
            pltpu.VMEM((QR, D_MODEL), jnp.bfloat16),
            pltpu.VMEM((N_DEV, QR, D_MODEL), jnp.bfloat16),
            pltpu.VMEM((HD * N_DEV, D_MODEL), jnp.bfloat16),
            pltpu.SemaphoreType.DMA((N_DEV,)),
            pltpu.SemaphoreType.DMA((N_DEV,)),
            pltpu.SemaphoreType.DMA((N_DEV,)),
            pltpu.SemaphoreType.DMA((N_DEV,)),
        ],
        compiler_params=pltpu.CompilerParams(collective_id=0),
    )(x, Wq, K_ext, V_ext, Wo)


# device time: 16780 ns/iter; 1.6620x vs baseline; 1.0962x over previous
import jax
import jax.numpy as jnp
from jax import lax
from jax.experimental import pallas as pl
from jax.experimental.pallas import tpu as pltpu

N_DEV = 4
B, SQ, SKV, DH = 2, 256, 256, 64
H_PER = 4
HD = H_PER * DH
D_MODEL = 512
ROWS = B * SQ


def kernel(x, Wq, K_ext, V_ext, Wo):
    def body(x_ref, wq_ref, k_ref, v_ref, wo_ref, out_ref,
             ctx_ref, rs_ref, wobf_ref, send_sems, recv_sems):
        my = lax.axis_index("i")

        barrier_sem = pltpu.get_barrier_semaphore()
        for r in range(1, N_DEV):
            pl.semaphore_signal(
                barrier_sem, inc=1,
                device_id=(lax.rem(my + r, N_DEV),),
                device_id_type=pl.DeviceIdType.MESH,
            )

        wq = wq_ref[:, pl.ds(my * HD, HD)]

        qb = lax.broadcasted_iota(jnp.int32, (SQ, SKV), 0) // 64
        kb = lax.broadcasted_iota(jnp.int32, (SQ, SKV), 1) // 64
        maskf = jnp.where(
            (qb == kb) | ((kb % 4) == (qb % 4)), 1.0, 0.0
        ).astype(jnp.float32)

        descs = {}
        for b in range(B):
            q = jnp.dot(x_ref[b], wq, preferred_element_type=jnp.float32)
            q = q * 0.125
            ctx_parts = []
            for h in range(H_PER):
                qh = q[:, h * DH:(h + 1) * DH]
                kh = k_ref[b, :, h, :]
                vh = v_ref[b, :, h, :]
                s = lax.dot_general(
                    qh, kh, (((1,), (1,)), ((), ())),
                    preferred_element_type=jnp.float32,
                )
                w = jnp.exp(s) * maskf
                rs = 1.0 / jnp.sum(w, axis=-1, keepdims=True)
                ctx_parts.append(
                    jnp.dot(w, vh, preferred_element_type=jnp.float32) * rs)
            ctx_ref[pl.ds(b * SQ, SQ)] = jnp.concatenate(
                ctx_parts, axis=1).astype(jnp.bfloat16)

            if b == 0:
                pl.semaphore_wait(barrier_sem, N_DEV - 1)

            for r in (2, 1, 3):
                desc = pltpu.make_async_remote_copy(
                    src_ref=ctx_ref.at[pl.ds(b * SQ, SQ)],
                    dst_ref=rs_ref.at[b, r],
                    send_sem=send_sems.at[b, r],
                    recv_sem=recv_sems.at[b, r],
                    device_id=(lax.rem(my + r, N_DEV),),
                    device_id_type=pl.DeviceIdType.MESH,
                )
                desc.start()
                descs[(b, r)] = desc

        wobf_ref[...] = wo_ref[...].astype(jnp.bfloat16)
        wo_mine = wobf_ref[pl.ds(my * HD, HD), :]

        for b in range(B):
            acc = jnp.dot(
                ctx_ref[pl.ds(b * SQ, SQ)], wo_mine,
                preferred_element_type=jnp.float32,
            )
            for r in (1, 3, 2):
                descs[(b, r)].wait_recv()
                sender = lax.rem(my + N_DEV - r, N_DEV)
                wo_s = wobf_ref[pl.ds(sender * HD, HD), :]
                acc = acc + jnp.dot(
                    rs_ref[b, r], wo_s, preferred_element_type=jnp.float32)
            out_ref[b] = acc

        for b in range(B):
            for r in (1, 2, 3):
                descs[(b, r)].wait_send()

    return pl.pallas_call(
        body,
        out_shape=jax.ShapeDtypeStruct((B, SQ, D_MODEL), jnp.float32),
        in_specs=[pl.BlockSpec(memory_space=pltpu.VMEM)] * 5,
        out_specs=pl.BlockSpec(memory_space=pltpu.VMEM),
        scratch_shapes=[
            pltpu.VMEM((ROWS, HD), jnp.bfloat16),
            pltpu.VMEM((B, N_DEV, SQ, HD), jnp.bfloat16),
            pltpu.VMEM((HD * N_DEV, D_MODEL), jnp.bfloat16),
            pltpu.SemaphoreType.DMA((B, N_DEV)),
            pltpu.SemaphoreType.DMA((B, N_DEV)),
        ],
        compiler_params=pltpu.CompilerParams(collective_id=0),
    )(x, Wq, K_ext, V_ext, Wo)
